# baseline (speedup 1.0000x reference)
import functools

import jax
import jax.numpy as jnp
from jax import lax
from jax.experimental import pallas as pl
from jax.experimental.pallas import tpu as pltpu
from jax.experimental.pallas import tpu_sc as plsc

E = 8
TOPK = 2
H = 2048
I = 1024
T = 4096
F = T * TOPK
RB = 128
S = F + E * RB
NRB = S // RB
N1 = 512
NN1 = I // N1
N2 = 1024
NN2 = H // N2

NC = 2
NS = 16
NW = NC * NS
LANES = 16
PW = S // NW
TW = T // NW
CH = 16

_SC_MESH = plsc.VectorSubcoreMesh(core_axis_name="c", subcore_axis_name="s")


def _dispatch_body(sp_hbm, x_hbm, ew_hbm, xs_hbm, ws_hbm,
                   sp_v, rows_v, ws_v, sem):
    wid = lax.axis_index("s") * NC + lax.axis_index("c")
    base = wid * PW
    pltpu.sync_copy(sp_hbm.at[pl.ds(base, PW)], sp_v)
    pltpu.async_copy(ew_hbm.at[sp_v], ws_v, sem).wait()
    pltpu.sync_copy(ws_v, ws_hbm.at[pl.ds(base, PW)])

    def chunk(c, carry):
        spv = sp_v[pl.ds(c * CH, CH)]
        tok = lax.shift_right_logical(spv, 1)
        pltpu.async_copy(x_hbm.at[tok], rows_v, sem).wait()
        pltpu.sync_copy(rows_v, xs_hbm.at[pl.ds(base + c * CH, CH)])
        return carry

    lax.fori_loop(0, PW // CH, chunk, 0)


_dispatch = pl.kernel(
    _dispatch_body,
    out_type=[
        jax.ShapeDtypeStruct((S, H), jnp.float32),
        jax.ShapeDtypeStruct((S,), jnp.float32),
    ],
    mesh=_SC_MESH,
    scratch_types=[
        pltpu.VMEM((PW,), jnp.int32),
        pltpu.VMEM((CH, H), jnp.float32),
        pltpu.VMEM((PW,), jnp.float32),
        pltpu.SemaphoreType.DMA,
    ],
)


def _combine_body(d0_hbm, d1_hbm, ys_hbm, out_hbm,
                  d0_v, d1_v, a_v, b_v, sem):
    wid = lax.axis_index("s") * NC + lax.axis_index("c")
    base = wid * TW
    pltpu.sync_copy(d0_hbm.at[pl.ds(base, TW)], d0_v)
    pltpu.sync_copy(d1_hbm.at[pl.ds(base, TW)], d1_v)

    def chunk(c, carry):
        ia = d0_v[pl.ds(c * CH, CH)]
        ib = d1_v[pl.ds(c * CH, CH)]
        cpa = pltpu.async_copy(ys_hbm.at[ia], a_v, sem)
        cpb = pltpu.async_copy(ys_hbm.at[ib], b_v, sem)
        cpa.wait()
        cpb.wait()

        def add16(k, carry2):
            r = k // (H // LANES)
            j = (k % (H // LANES)) * LANES
            a_v[r, pl.ds(j, LANES)] = (a_v[r, pl.ds(j, LANES)]
                                       + b_v[r, pl.ds(j, LANES)])
            return carry2

        lax.fori_loop(0, CH * (H // LANES), add16, 0)
        pltpu.sync_copy(a_v, out_hbm.at[pl.ds(base + c * CH, CH)])
        return carry

    lax.fori_loop(0, TW // CH, chunk, 0)


_combine = pl.kernel(
    _combine_body,
    out_type=jax.ShapeDtypeStruct((T, H), jnp.float32),
    mesh=_SC_MESH,
    scratch_types=[
        pltpu.VMEM((TW,), jnp.int32),
        pltpu.VMEM((TW,), jnp.int32),
        pltpu.VMEM((CH, H), jnp.float32),
        pltpu.VMEM((CH, H), jnp.float32),
        pltpu.SemaphoreType.DMA,
    ],
)


def _ffn1_body(be_ref, xs_ref, w1g_ref, w1u_ref, ws_ref, act_ref):
    del be_ref
    xb = xs_ref[...]
    g = lax.dot_general(xb, w1g_ref[0], (((1,), (1,)), ((), ())),
                        preferred_element_type=jnp.float32)
    u = lax.dot_general(xb, w1u_ref[0], (((1,), (1,)), ((), ())),
                        preferred_element_type=jnp.float32)
    act_ref[...] = (g * jax.nn.sigmoid(g)) * u * ws_ref[:, :1]


def _ffn2_body(be_ref, act_ref, w2_ref, ys_ref):
    del be_ref
    ys_ref[...] = lax.dot_general(act_ref[...], w2_ref[0],
                                  (((1,), (1,)), ((), ())),
                                  preferred_element_type=jnp.float32)


_ffn1 = pl.pallas_call(
    _ffn1_body,
    grid_spec=pltpu.PrefetchScalarGridSpec(
        num_scalar_prefetch=1,
        grid=(NN1, NRB),
        in_specs=[
            pl.BlockSpec((RB, H), lambda n, i, be: (i, 0)),
            pl.BlockSpec((1, N1, H), lambda n, i, be: (be[i], n, 0)),
            pl.BlockSpec((1, N1, H), lambda n, i, be: (be[i], n, 0)),
            pl.BlockSpec((RB, 128), lambda n, i, be: (i, 0)),
        ],
        out_specs=pl.BlockSpec((RB, N1), lambda n, i, be: (i, n)),
    ),
    out_shape=jax.ShapeDtypeStruct((S, I), jnp.float32),
)

_ffn2 = pl.pallas_call(
    _ffn2_body,
    grid_spec=pltpu.PrefetchScalarGridSpec(
        num_scalar_prefetch=1,
        grid=(NN2, NRB),
        in_specs=[
            pl.BlockSpec((RB, I), lambda n, i, be: (i, 0)),
            pl.BlockSpec((1, N2, I), lambda n, i, be: (be[i], n, 0)),
        ],
        out_specs=pl.BlockSpec((RB, N2), lambda n, i, be: (i, n)),
    ),
    out_shape=jax.ShapeDtypeStruct((S, H), jnp.float32),
)


def kernel(x, expert_weights, expert_indices, top_k, w1_weight, w2_weight):
    del top_k
    fe = expert_indices.reshape(F).astype(jnp.int32)
    oh = (fe[:, None] == jnp.arange(E, dtype=jnp.int32)[None, :]).astype(jnp.int32)
    csum = jnp.cumsum(oh, axis=0)
    counts = csum[-1]
    rank = jnp.sum((csum - oh) * oh, axis=1)
    pc = ((counts + RB - 1) // RB) * RB
    cum = jnp.cumsum(pc)
    poffs = cum - pc
    dest = poffs[fe] + rank
    sorted_pair = jnp.zeros((S,), jnp.int32).at[dest].set(
        jnp.arange(F, dtype=jnp.int32))
    block_expert = jnp.minimum(
        jnp.searchsorted(cum, jnp.arange(NRB, dtype=jnp.int32) * RB,
                         side="right"),
        E - 1).astype(jnp.int32)
    dtk = dest.reshape(T, TOPK)
    d0 = dtk[:, 0].astype(jnp.int32)
    d1 = dtk[:, 1].astype(jnp.int32)
    ewf = expert_weights.reshape(F).astype(jnp.float32)

    xs, ws = _dispatch(sorted_pair, x, ewf)
    wsb = jnp.broadcast_to(ws[:, None], (S, 128))
    w1g = w1_weight[:, :I, :]
    w1u = w1_weight[:, I:, :]
    act = _ffn1(block_expert, xs, w1g, w1u, wsb)
    ys = _ffn2(block_expert, act, w2_weight)
    return _combine(d0, d1, ys)


# speedup vs baseline: 1.0563x; 1.0563x over previous
import jax
import jax.numpy as jnp
from jax import lax
from jax.experimental import pallas as pl
from jax.experimental.pallas import tpu as pltpu
from jax.experimental.pallas import tpu_sc as plsc

E = 8
TOPK = 2
H = 2048
I = 1024
T = 4096
F = T * TOPK
RB = 128
S = F + E * RB
NRB = S // RB
N1 = 1024
NN1 = I // N1
N2 = 2048
NN2 = H // N2

NC = 2
NS = 16
NW = NC * NS
LANES = 16
PW = S // NW
TW = T // NW

CH = 16
DBUF = 3
DCHUNKS = PW // CH
DROUNDS = DCHUNKS // DBUF

CCH = 8
CCHUNKS = TW // CCH
CROUNDS = CCHUNKS // 2

_SC_MESH = plsc.VectorSubcoreMesh(core_axis_name="c", subcore_axis_name="s")


def _dispatch_body(sp_hbm, x_hbm, ew_hbm, xs_hbm, ws_hbm,
                   sp_v, r0, r1, r2, ws_v,
                   si0, si1, si2, so0, so1, so2):
    wid = lax.axis_index("s") * NC + lax.axis_index("c")
    base = wid * PW
    pltpu.sync_copy(sp_hbm.at[pl.ds(base, PW)], sp_v)
    pltpu.async_copy(ew_hbm.at[sp_v], ws_v, si0).wait()
    pltpu.sync_copy(ws_v, ws_hbm.at[pl.ds(base, PW)])

    bufs = (r0, r1, r2)
    sin = (si0, si1, si2)
    sout = (so0, so1, so2)

    def fire_in(b, c):
        spv = sp_v[pl.ds(c * CH, CH)]
        tok = lax.shift_right_logical(spv, 1)
        pltpu.async_copy(x_hbm.at[tok], bufs[b], sin[b])

    def wait_in(b):
        pltpu.make_async_copy(x_hbm.at[pl.ds(0, CH)], bufs[b], sin[b]).wait()

    def fire_out(b, c):
        pltpu.async_copy(bufs[b], xs_hbm.at[pl.ds(base + c * CH, CH)],
                         sout[b])

    def wait_out(b):
        pltpu.make_async_copy(bufs[b], xs_hbm.at[pl.ds(0, CH)],
                              sout[b]).wait()

    for b in range(DBUF):
        fire_in(b, b)

    def round_body(r, carry):
        for b in range(DBUF):
            wait_in(b)
            fire_out(b, r * DBUF + b)
        for b in range(DBUF):
            @pl.when(r < DROUNDS - 1)
            def _():
                wait_out(b)
                fire_in(b, (r + 1) * DBUF + b)
        return carry

    lax.fori_loop(0, DROUNDS, round_body, 0)
    for b in range(DBUF):
        wait_out(b)


_dispatch = pl.kernel(
    _dispatch_body,
    out_type=[
        jax.ShapeDtypeStruct((S, H), jnp.float32),
        jax.ShapeDtypeStruct((S,), jnp.float32),
    ],
    mesh=_SC_MESH,
    scratch_types=[
        pltpu.VMEM((PW,), jnp.int32),
        pltpu.VMEM((CH, H), jnp.float32),
        pltpu.VMEM((CH, H), jnp.float32),
        pltpu.VMEM((CH, H), jnp.float32),
        pltpu.VMEM((PW,), jnp.float32),
        pltpu.SemaphoreType.DMA,
        pltpu.SemaphoreType.DMA,
        pltpu.SemaphoreType.DMA,
        pltpu.SemaphoreType.DMA,
        pltpu.SemaphoreType.DMA,
        pltpu.SemaphoreType.DMA,
    ],
)


def _combine_body(d0_hbm, d1_hbm, ys_hbm, out_hbm,
                  d0_v, d1_v, a0, b0, a1, b1, si0, si1, so0, so1):
    wid = lax.axis_index("s") * NC + lax.axis_index("c")
    base = wid * TW
    pltpu.sync_copy(d0_hbm.at[wid], d0_v)
    pltpu.sync_copy(d1_hbm.at[wid], d1_v)

    A = (a0, a1)
    B = (b0, b1)
    sin = (si0, si1)
    sout = (so0, so1)

    def fire_in(b, c):
        pltpu.async_copy(ys_hbm.at[d0_v.at[c]], A[b], sin[b])
        pltpu.async_copy(ys_hbm.at[d1_v.at[c]], B[b], sin[b])

    def wait_in(b):
        pltpu.make_async_copy(ys_hbm.at[pl.ds(0, CCH)], A[b], sin[b]).wait()
        pltpu.make_async_copy(ys_hbm.at[pl.ds(0, CCH)], B[b], sin[b]).wait()

    def compute(b):
        av, bv = A[b], B[b]

        def add16(k, carry2):
            r = k // (H // LANES)
            j = (k % (H // LANES)) * LANES
            av[r, pl.ds(j, LANES)] = (av[r, pl.ds(j, LANES)]
                                      + bv[r, pl.ds(j, LANES)])
            return carry2

        lax.fori_loop(0, CCH * (H // LANES), add16, 0)

    def fire_out(b, c):
        pltpu.async_copy(A[b], out_hbm.at[pl.ds(base + c * CCH, CCH)],
                         sout[b])

    def wait_out(b):
        pltpu.make_async_copy(A[b], out_hbm.at[pl.ds(0, CCH)],
                              sout[b]).wait()

    fire_in(0, 0)

    def round_body(r, carry):
        c = r * 2
        wait_in(0)

        @pl.when(r > 0)
        def _():
            wait_out(1)

        fire_in(1, c + 1)
        compute(0)
        fire_out(0, c)
        wait_in(1)

        @pl.when(r < CROUNDS - 1)
        def _():
            wait_out(0)
            fire_in(0, c + 2)

        compute(1)
        fire_out(1, c + 1)
        return carry

    lax.fori_loop(0, CROUNDS, round_body, 0)
    wait_out(0)
    wait_out(1)


_combine = pl.kernel(
    _combine_body,
    out_type=jax.ShapeDtypeStruct((T, H), jnp.float32),
    mesh=_SC_MESH,
    scratch_types=[
        pltpu.VMEM((CCHUNKS, CCH), jnp.int32),
        pltpu.VMEM((CCHUNKS, CCH), jnp.int32),
        pltpu.VMEM((CCH, H), jnp.float32),
        pltpu.VMEM((CCH, H), jnp.float32),
        pltpu.VMEM((CCH, H), jnp.float32),
        pltpu.VMEM((CCH, H), jnp.float32),
        pltpu.SemaphoreType.DMA,
        pltpu.SemaphoreType.DMA,
        pltpu.SemaphoreType.DMA,
        pltpu.SemaphoreType.DMA,
    ],
)


def _ffn1_body(be_ref, xs_ref, w1g_ref, w1u_ref, ws_ref, act_ref):
    del be_ref
    xb = xs_ref[...].astype(jnp.bfloat16)
    g = lax.dot_general(xb, w1g_ref[0], (((1,), (1,)), ((), ())),
                        preferred_element_type=jnp.float32)
    u = lax.dot_general(xb, w1u_ref[0], (((1,), (1,)), ((), ())),
                        preferred_element_type=jnp.float32)
    act = (g * jax.nn.sigmoid(g)) * u * ws_ref[:, :1]
    act_ref[...] = act.astype(jnp.bfloat16)


def _ffn2_body(be_ref, act_ref, w2_ref, ys_ref):
    del be_ref
    ys_ref[...] = lax.dot_general(act_ref[...], w2_ref[0],
                                  (((1,), (1,)), ((), ())),
                                  preferred_element_type=jnp.float32)


_ffn1 = pl.pallas_call(
    _ffn1_body,
    grid_spec=pltpu.PrefetchScalarGridSpec(
        num_scalar_prefetch=1,
        grid=(NN1, NRB),
        in_specs=[
            pl.BlockSpec((RB, H), lambda n, i, be: (i, 0)),
            pl.BlockSpec((1, N1, H), lambda n, i, be: (be[i], n, 0)),
            pl.BlockSpec((1, N1, H), lambda n, i, be: (be[i], n, 0)),
            pl.BlockSpec((RB, 128), lambda n, i, be: (i, 0)),
        ],
        out_specs=pl.BlockSpec((RB, N1), lambda n, i, be: (i, n)),
    ),
    out_shape=jax.ShapeDtypeStruct((S, I), jnp.bfloat16),
)

_ffn2 = pl.pallas_call(
    _ffn2_body,
    grid_spec=pltpu.PrefetchScalarGridSpec(
        num_scalar_prefetch=1,
        grid=(NN2, NRB),
        in_specs=[
            pl.BlockSpec((RB, I), lambda n, i, be: (i, 0)),
            pl.BlockSpec((1, N2, I), lambda n, i, be: (be[i], n, 0)),
        ],
        out_specs=pl.BlockSpec((RB, N2), lambda n, i, be: (i, n)),
    ),
    out_shape=jax.ShapeDtypeStruct((S, H), jnp.float32),
)


def kernel(x, expert_weights, expert_indices, top_k, w1_weight, w2_weight):
    del top_k
    fe = expert_indices.reshape(F).astype(jnp.int32)
    oh = (fe[:, None] == jnp.arange(E, dtype=jnp.int32)[None, :]).astype(jnp.int32)
    csum = jnp.cumsum(oh, axis=0)
    counts = csum[-1]
    rank = jnp.sum((csum - oh) * oh, axis=1)
    pc = ((counts + RB - 1) // RB) * RB
    cum = jnp.cumsum(pc)
    poffs = cum - pc
    dest = poffs[fe] + rank
    sorted_pair = jnp.zeros((S,), jnp.int32).at[dest].set(
        jnp.arange(F, dtype=jnp.int32))
    block_expert = jnp.minimum(
        jnp.searchsorted(cum, jnp.arange(NRB, dtype=jnp.int32) * RB,
                         side="right"),
        E - 1).astype(jnp.int32)
    dtk = dest.reshape(T, TOPK)
    d0 = dtk[:, 0].astype(jnp.int32).reshape(NW, CCHUNKS, CCH)
    d1 = dtk[:, 1].astype(jnp.int32).reshape(NW, CCHUNKS, CCH)
    ewf = expert_weights.reshape(F).astype(jnp.float32)

    xs, ws = _dispatch(sorted_pair, x, ewf)
    wsb = jnp.broadcast_to(ws[:, None], (S, 128))
    w1g = w1_weight[:, :I, :].astype(jnp.bfloat16)
    w1u = w1_weight[:, I:, :].astype(jnp.bfloat16)
    act = _ffn1(block_expert, xs, w1g, w1u, wsb)
    ys = _ffn2(block_expert, act, w2_weight.astype(jnp.bfloat16))
    return _combine(d0, d1, ys)


# speedup vs baseline: 1.2226x; 1.1574x over previous
import jax
import jax.numpy as jnp
from jax import lax
from jax.experimental import pallas as pl
from jax.experimental.pallas import tpu as pltpu
from jax.experimental.pallas import tpu_sc as plsc

E = 8
TOPK = 2
H = 2048
I = 1024
T = 4096
F = T * TOPK
RB = 128
S = F + E * RB
NRB = S // RB
N1 = 1024
NN1 = I // N1
N2 = 2048
NN2 = H // N2

NC = 2
NS = 16
NW = NC * NS
LANES = 16
PW = S // NW
TW = T // NW

CH = 16
DBUF = 3
DCHUNKS = PW // CH
DROUNDS = DCHUNKS // DBUF

CCH = 8
CCHUNKS = TW // CCH
CROUNDS = CCHUNKS // 2

_SC_MESH = plsc.VectorSubcoreMesh(core_axis_name="c", subcore_axis_name="s")


def _dispatch_body(sp_hbm, x_hbm, ew_hbm, xs_hbm, ws_hbm,
                   sp_v, r0, r1, r2, ws_v,
                   si0, si1, si2, so0, so1, so2):
    wid = lax.axis_index("s") * NC + lax.axis_index("c")
    base = wid * PW
    pltpu.sync_copy(sp_hbm.at[pl.ds(base, PW)], sp_v)
    pltpu.async_copy(ew_hbm.at[sp_v], ws_v, si0).wait()
    pltpu.sync_copy(ws_v, ws_hbm.at[pl.ds(base, PW)])

    bufs = (r0, r1, r2)
    sin = (si0, si1, si2)
    sout = (so0, so1, so2)

    def fire_in(b, c):
        spv = sp_v[pl.ds(c * CH, CH)]
        tok = lax.shift_right_logical(spv, 1)
        pltpu.async_copy(x_hbm.at[tok], bufs[b], sin[b])

    def wait_in(b):
        pltpu.make_async_copy(x_hbm.at[pl.ds(0, CH)], bufs[b], sin[b]).wait()

    def fire_out(b, c):
        pltpu.async_copy(bufs[b], xs_hbm.at[pl.ds(base + c * CH, CH)],
                         sout[b])

    def wait_out(b):
        pltpu.make_async_copy(bufs[b], xs_hbm.at[pl.ds(0, CH)],
                              sout[b]).wait()

    for b in range(DBUF):
        fire_in(b, b)

    def round_body(r, carry):
        for b in range(DBUF):
            wait_in(b)
            fire_out(b, r * DBUF + b)
        for b in range(DBUF):
            @pl.when(r < DROUNDS - 1)
            def _():
                wait_out(b)
                fire_in(b, (r + 1) * DBUF + b)
        return carry

    lax.fori_loop(0, DROUNDS, round_body, 0)
    for b in range(DBUF):
        wait_out(b)


_dispatch = pl.kernel(
    _dispatch_body,
    out_type=[
        jax.ShapeDtypeStruct((S, H), jnp.float32),
        jax.ShapeDtypeStruct((S,), jnp.float32),
    ],
    mesh=_SC_MESH,
    scratch_types=[
        pltpu.VMEM((PW,), jnp.int32),
        pltpu.VMEM((CH, H), jnp.float32),
        pltpu.VMEM((CH, H), jnp.float32),
        pltpu.VMEM((CH, H), jnp.float32),
        pltpu.VMEM((PW,), jnp.float32),
        pltpu.SemaphoreType.DMA,
        pltpu.SemaphoreType.DMA,
        pltpu.SemaphoreType.DMA,
        pltpu.SemaphoreType.DMA,
        pltpu.SemaphoreType.DMA,
        pltpu.SemaphoreType.DMA,
    ],
)


def _combine_body(d0_hbm, d1_hbm, ys_hbm, out_hbm,
                  d0_v, d1_v, a0, b0, a1, b1, si0, si1, so0, so1):
    wid = lax.axis_index("s") * NC + lax.axis_index("c")
    base = wid * TW
    pltpu.sync_copy(d0_hbm.at[wid], d0_v)
    pltpu.sync_copy(d1_hbm.at[wid], d1_v)

    A = (a0, a1)
    B = (b0, b1)
    sin = (si0, si1)
    sout = (so0, so1)

    def fire_in(b, c):
        pltpu.async_copy(ys_hbm.at[d0_v.at[c]], A[b], sin[b])
        pltpu.async_copy(ys_hbm.at[d1_v.at[c]], B[b], sin[b])

    def wait_in(b):
        pltpu.make_async_copy(ys_hbm.at[pl.ds(0, CCH)], A[b], sin[b]).wait()
        pltpu.make_async_copy(ys_hbm.at[pl.ds(0, CCH)], B[b], sin[b]).wait()

    def compute(b):
        av, bv = A[b], B[b]

        def add16(k, carry2):
            r = k // (H // LANES)
            j = (k % (H // LANES)) * LANES
            av[r, pl.ds(j, LANES)] = (av[r, pl.ds(j, LANES)]
                                      + bv[r, pl.ds(j, LANES)])
            return carry2

        lax.fori_loop(0, CCH * (H // LANES), add16, 0)

    def fire_out(b, c):
        pltpu.async_copy(A[b], out_hbm.at[pl.ds(base + c * CCH, CCH)],
                         sout[b])

    def wait_out(b):
        pltpu.make_async_copy(A[b], out_hbm.at[pl.ds(0, CCH)],
                              sout[b]).wait()

    fire_in(0, 0)

    def round_body(r, carry):
        c = r * 2
        wait_in(0)

        @pl.when(r > 0)
        def _():
            wait_out(1)

        fire_in(1, c + 1)
        compute(0)
        fire_out(0, c)
        wait_in(1)

        @pl.when(r < CROUNDS - 1)
        def _():
            wait_out(0)
            fire_in(0, c + 2)

        compute(1)
        fire_out(1, c + 1)
        return carry

    lax.fori_loop(0, CROUNDS, round_body, 0)
    wait_out(0)
    wait_out(1)


_combine = pl.kernel(
    _combine_body,
    out_type=jax.ShapeDtypeStruct((T, H), jnp.float32),
    mesh=_SC_MESH,
    scratch_types=[
        pltpu.VMEM((CCHUNKS, CCH), jnp.int32),
        pltpu.VMEM((CCHUNKS, CCH), jnp.int32),
        pltpu.VMEM((CCH, H), jnp.float32),
        pltpu.VMEM((CCH, H), jnp.float32),
        pltpu.VMEM((CCH, H), jnp.float32),
        pltpu.VMEM((CCH, H), jnp.float32),
        pltpu.SemaphoreType.DMA,
        pltpu.SemaphoreType.DMA,
        pltpu.SemaphoreType.DMA,
        pltpu.SemaphoreType.DMA,
    ],
)


def _ffn1_body(be_ref, xs_ref, w1g_ref, w1u_ref, ws_ref, act_ref):
    del be_ref
    xb = xs_ref[...].astype(jnp.bfloat16)
    w1g = w1g_ref[0].astype(jnp.bfloat16)
    w1u = w1u_ref[0].astype(jnp.bfloat16)
    g = lax.dot_general(xb, w1g, (((1,), (1,)), ((), ())),
                        preferred_element_type=jnp.float32)
    u = lax.dot_general(xb, w1u, (((1,), (1,)), ((), ())),
                        preferred_element_type=jnp.float32)
    act = (g * jax.nn.sigmoid(g)) * u * ws_ref[:, :1]
    act_ref[...] = act.astype(jnp.bfloat16)


def _ffn2_body(be_ref, act_ref, w2_ref, ys_ref):
    del be_ref
    ys_ref[...] = lax.dot_general(act_ref[...], w2_ref[0].astype(jnp.bfloat16),
                                  (((1,), (1,)), ((), ())),
                                  preferred_element_type=jnp.float32)


_ffn1 = pl.pallas_call(
    _ffn1_body,
    grid_spec=pltpu.PrefetchScalarGridSpec(
        num_scalar_prefetch=1,
        grid=(NN1, NRB),
        in_specs=[
            pl.BlockSpec((RB, H), lambda n, i, be: (i, 0)),
            pl.BlockSpec((1, N1, H), lambda n, i, be: (be[i], n, 0)),
            pl.BlockSpec((1, N1, H), lambda n, i, be: (be[i], NN1 + n, 0)),
            pl.BlockSpec((RB, 128), lambda n, i, be: (i, 0)),
        ],
        out_specs=pl.BlockSpec((RB, N1), lambda n, i, be: (i, n)),
    ),
    out_shape=jax.ShapeDtypeStruct((S, I), jnp.bfloat16),
)

_ffn2 = pl.pallas_call(
    _ffn2_body,
    grid_spec=pltpu.PrefetchScalarGridSpec(
        num_scalar_prefetch=1,
        grid=(NN2, NRB),
        in_specs=[
            pl.BlockSpec((RB, I), lambda n, i, be: (i, 0)),
            pl.BlockSpec((1, N2, I), lambda n, i, be: (be[i], n, 0)),
        ],
        out_specs=pl.BlockSpec((RB, N2), lambda n, i, be: (i, n)),
    ),
    out_shape=jax.ShapeDtypeStruct((S, H), jnp.float32),
)


def kernel(x, expert_weights, expert_indices, top_k, w1_weight, w2_weight):
    del top_k
    fe = expert_indices.reshape(F).astype(jnp.int32)
    oh = (fe[:, None] == jnp.arange(E, dtype=jnp.int32)[None, :]).astype(jnp.int32)
    csum = jnp.cumsum(oh, axis=0)
    counts = csum[-1]
    rank = jnp.sum((csum - oh) * oh, axis=1)
    pc = ((counts + RB - 1) // RB) * RB
    cum = jnp.cumsum(pc)
    poffs = cum - pc
    dest = poffs[fe] + rank
    sorted_pair = jnp.zeros((S,), jnp.int32).at[dest].set(
        jnp.arange(F, dtype=jnp.int32))
    block_expert = jnp.minimum(
        jnp.searchsorted(cum, jnp.arange(NRB, dtype=jnp.int32) * RB,
                         side="right"),
        E - 1).astype(jnp.int32)
    dtk = dest.reshape(T, TOPK)
    d0 = dtk[:, 0].astype(jnp.int32).reshape(NW, CCHUNKS, CCH)
    d1 = dtk[:, 1].astype(jnp.int32).reshape(NW, CCHUNKS, CCH)
    ewf = expert_weights.reshape(F).astype(jnp.float32)

    xs, ws = _dispatch(sorted_pair, x, ewf)
    wsb = jnp.broadcast_to(ws[:, None], (S, 128))
    act = _ffn1(block_expert, xs, w1_weight, w1_weight, wsb)
    ys = _ffn2(block_expert, act, w2_weight)
    return _combine(d0, d1, ys)


# speedup vs baseline: 1.2266x; 1.0033x over previous
import jax
import jax.numpy as jnp
from jax import lax
from jax.experimental import pallas as pl
from jax.experimental.pallas import tpu as pltpu
from jax.experimental.pallas import tpu_sc as plsc

E = 8
TOPK = 2
H = 2048
I = 1024
T = 4096
F = T * TOPK
RB = 128
S = F + E * RB
NRB = S // RB
N1 = 1024
NN1 = I // N1
N2 = 2048
NN2 = H // N2

NC = 2
NS = 16
NW = NC * NS
LANES = 16
PW = S // NW
TW = T // NW

CH = 16
DBUF = 3
DCHUNKS = PW // CH
DROUNDS = DCHUNKS // DBUF

CCH = 8
CCHUNKS = TW // CCH
CROUNDS = CCHUNKS // 2

_SC_MESH = plsc.VectorSubcoreMesh(core_axis_name="c", subcore_axis_name="s")


def _dispatch_body(sp_hbm, x_hbm, ew_hbm, xs_hbm, ws_hbm,
                   sp_v, r0, r1, r2, ws_v,
                   si0, si1, si2, so0, so1, so2):
    wid = lax.axis_index("s") * NC + lax.axis_index("c")
    base = wid * PW
    pltpu.sync_copy(sp_hbm.at[pl.ds(base, PW)], sp_v)
    pltpu.async_copy(ew_hbm.at[sp_v], ws_v, si0).wait()
    pltpu.sync_copy(ws_v, ws_hbm.at[pl.ds(base, PW)])

    bufs = (r0, r1, r2)
    sin = (si0, si1, si2)
    sout = (so0, so1, so2)

    def fire_in(b, c):
        spv = sp_v[pl.ds(c * CH, CH)]
        tok = lax.shift_right_logical(spv, 1)
        pltpu.async_copy(x_hbm.at[tok], bufs[b], sin[b])

    def wait_in(b):
        pltpu.make_async_copy(x_hbm.at[pl.ds(0, CH)], bufs[b], sin[b]).wait()

    def fire_out(b, c):
        pltpu.async_copy(bufs[b], xs_hbm.at[pl.ds(base + c * CH, CH)],
                         sout[b])

    def wait_out(b):
        pltpu.make_async_copy(bufs[b], xs_hbm.at[pl.ds(0, CH)],
                              sout[b]).wait()

    for b in range(DBUF):
        fire_in(b, b)

    def round_body(r, carry):
        for b in range(DBUF):
            wait_in(b)
            fire_out(b, r * DBUF + b)
        for b in range(DBUF):
            @pl.when(r < DROUNDS - 1)
            def _():
                wait_out(b)
                fire_in(b, (r + 1) * DBUF + b)
        return carry

    lax.fori_loop(0, DROUNDS, round_body, 0)
    for b in range(DBUF):
        wait_out(b)


_dispatch = pl.kernel(
    _dispatch_body,
    out_type=[
        jax.ShapeDtypeStruct((S, H), jnp.float32),
        jax.ShapeDtypeStruct((S,), jnp.float32),
    ],
    mesh=_SC_MESH,
    scratch_types=[
        pltpu.VMEM((PW,), jnp.int32),
        pltpu.VMEM((CH, H), jnp.float32),
        pltpu.VMEM((CH, H), jnp.float32),
        pltpu.VMEM((CH, H), jnp.float32),
        pltpu.VMEM((PW,), jnp.float32),
        pltpu.SemaphoreType.DMA,
        pltpu.SemaphoreType.DMA,
        pltpu.SemaphoreType.DMA,
        pltpu.SemaphoreType.DMA,
        pltpu.SemaphoreType.DMA,
        pltpu.SemaphoreType.DMA,
    ],
)


def _combine_body(d0_hbm, d1_hbm, ys_hbm, out_hbm,
                  d0_v, d1_v, a0, b0, a1, b1, si0, si1, so0, so1):
    wid = lax.axis_index("s") * NC + lax.axis_index("c")
    base = wid * TW
    pltpu.sync_copy(d0_hbm.at[wid], d0_v)
    pltpu.sync_copy(d1_hbm.at[wid], d1_v)

    A = (a0, a1)
    B = (b0, b1)
    sin = (si0, si1)
    sout = (so0, so1)

    def fire_in(b, c):
        pltpu.async_copy(ys_hbm.at[d0_v.at[c]], A[b], sin[b])
        pltpu.async_copy(ys_hbm.at[d1_v.at[c]], B[b], sin[b])

    def wait_in(b):
        pltpu.make_async_copy(ys_hbm.at[pl.ds(0, CCH)], A[b], sin[b]).wait()
        pltpu.make_async_copy(ys_hbm.at[pl.ds(0, CCH)], B[b], sin[b]).wait()

    def compute(b):
        av, bv = A[b], B[b]

        def add16(k, carry2):
            r = k // (H // LANES)
            j = (k % (H // LANES)) * LANES
            av[r, pl.ds(j, LANES)] = (av[r, pl.ds(j, LANES)]
                                      + bv[r, pl.ds(j, LANES)])
            return carry2

        lax.fori_loop(0, CCH * (H // LANES), add16, 0)

    def fire_out(b, c):
        pltpu.async_copy(A[b], out_hbm.at[pl.ds(base + c * CCH, CCH)],
                         sout[b])

    def wait_out(b):
        pltpu.make_async_copy(A[b], out_hbm.at[pl.ds(0, CCH)],
                              sout[b]).wait()

    fire_in(0, 0)

    def round_body(r, carry):
        c = r * 2
        wait_in(0)

        @pl.when(r > 0)
        def _():
            wait_out(1)

        fire_in(1, c + 1)
        compute(0)
        fire_out(0, c)
        wait_in(1)

        @pl.when(r < CROUNDS - 1)
        def _():
            wait_out(0)
            fire_in(0, c + 2)

        compute(1)
        fire_out(1, c + 1)
        return carry

    lax.fori_loop(0, CROUNDS, round_body, 0)
    wait_out(0)
    wait_out(1)


_combine = pl.kernel(
    _combine_body,
    out_type=jax.ShapeDtypeStruct((T, H), jnp.float32),
    mesh=_SC_MESH,
    scratch_types=[
        pltpu.VMEM((CCHUNKS, CCH), jnp.int32),
        pltpu.VMEM((CCHUNKS, CCH), jnp.int32),
        pltpu.VMEM((CCH, H), jnp.float32),
        pltpu.VMEM((CCH, H), jnp.float32),
        pltpu.VMEM((CCH, H), jnp.float32),
        pltpu.VMEM((CCH, H), jnp.float32),
        pltpu.SemaphoreType.DMA,
        pltpu.SemaphoreType.DMA,
        pltpu.SemaphoreType.DMA,
        pltpu.SemaphoreType.DMA,
    ],
)


def _ffn1_body(be_ref, xs_ref, w1g_ref, w1u_ref, ws_ref, act_ref,
               w1g_s, w1u_s):
    i = pl.program_id(1)
    changed = (i == 0) | (be_ref[i] != be_ref[jnp.maximum(i - 1, 0)])

    @pl.when(changed)
    def _():
        w1g_s[...] = w1g_ref[0].astype(jnp.bfloat16)
        w1u_s[...] = w1u_ref[0].astype(jnp.bfloat16)

    xb = xs_ref[...].astype(jnp.bfloat16)
    g = lax.dot_general(xb, w1g_s[...], (((1,), (1,)), ((), ())),
                        preferred_element_type=jnp.float32)
    u = lax.dot_general(xb, w1u_s[...], (((1,), (1,)), ((), ())),
                        preferred_element_type=jnp.float32)
    act = (g * jax.nn.sigmoid(g)) * u * ws_ref[:, :1]
    act_ref[...] = act.astype(jnp.bfloat16)


def _ffn2_body(be_ref, act_ref, w2_ref, ys_ref, w2_s):
    i = pl.program_id(1)
    changed = (i == 0) | (be_ref[i] != be_ref[jnp.maximum(i - 1, 0)])

    @pl.when(changed)
    def _():
        w2_s[...] = w2_ref[0].astype(jnp.bfloat16)

    ys_ref[...] = lax.dot_general(act_ref[...], w2_s[...],
                                  (((1,), (1,)), ((), ())),
                                  preferred_element_type=jnp.float32)


_ffn1 = pl.pallas_call(
    _ffn1_body,
    grid_spec=pltpu.PrefetchScalarGridSpec(
        num_scalar_prefetch=1,
        grid=(NN1, NRB),
        in_specs=[
            pl.BlockSpec((RB, H), lambda n, i, be: (i, 0)),
            pl.BlockSpec((1, N1, H), lambda n, i, be: (be[i], n, 0)),
            pl.BlockSpec((1, N1, H), lambda n, i, be: (be[i], NN1 + n, 0)),
            pl.BlockSpec((RB, 128), lambda n, i, be: (i, 0)),
        ],
        out_specs=pl.BlockSpec((RB, N1), lambda n, i, be: (i, n)),
        scratch_shapes=[
            pltpu.VMEM((N1, H), jnp.bfloat16),
            pltpu.VMEM((N1, H), jnp.bfloat16),
        ],
    ),
    out_shape=jax.ShapeDtypeStruct((S, I), jnp.bfloat16),
)

_ffn2 = pl.pallas_call(
    _ffn2_body,
    grid_spec=pltpu.PrefetchScalarGridSpec(
        num_scalar_prefetch=1,
        grid=(NN2, NRB),
        in_specs=[
            pl.BlockSpec((RB, I), lambda n, i, be: (i, 0)),
            pl.BlockSpec((1, N2, I), lambda n, i, be: (be[i], n, 0)),
        ],
        out_specs=pl.BlockSpec((RB, N2), lambda n, i, be: (i, n)),
        scratch_shapes=[
            pltpu.VMEM((N2, I), jnp.bfloat16),
        ],
    ),
    out_shape=jax.ShapeDtypeStruct((S, H), jnp.float32),
)


def kernel(x, expert_weights, expert_indices, top_k, w1_weight, w2_weight):
    del top_k
    fe = expert_indices.reshape(F).astype(jnp.int32)
    oh = (fe[:, None] == jnp.arange(E, dtype=jnp.int32)[None, :]).astype(jnp.int32)
    csum = jnp.cumsum(oh, axis=0)
    counts = csum[-1]
    rank = jnp.sum((csum - oh) * oh, axis=1)
    pc = ((counts + RB - 1) // RB) * RB
    cum = jnp.cumsum(pc)
    poffs = cum - pc
    dest = poffs[fe] + rank
    sorted_pair = jnp.zeros((S,), jnp.int32).at[dest].set(
        jnp.arange(F, dtype=jnp.int32))
    block_expert = jnp.minimum(
        jnp.searchsorted(cum, jnp.arange(NRB, dtype=jnp.int32) * RB,
                         side="right"),
        E - 1).astype(jnp.int32)
    dtk = dest.reshape(T, TOPK)
    d0 = dtk[:, 0].astype(jnp.int32).reshape(NW, CCHUNKS, CCH)
    d1 = dtk[:, 1].astype(jnp.int32).reshape(NW, CCHUNKS, CCH)
    ewf = expert_weights.reshape(F).astype(jnp.float32)

    xs, ws = _dispatch(sorted_pair, x, ewf)
    wsb = jnp.broadcast_to(ws[:, None], (S, 128))
    act = _ffn1(block_expert, xs, w1_weight, w1_weight, wsb)
    ys = _ffn2(block_expert, act, w2_weight)
    return _combine(d0, d1, ys)


# speedup vs baseline: 1.4286x; 1.1646x over previous
import jax
import jax.numpy as jnp
from jax import lax
from jax.experimental import pallas as pl
from jax.experimental.pallas import tpu as pltpu
from jax.experimental.pallas import tpu_sc as plsc

E = 8
TOPK = 2
H = 2048
I = 1024
T = 4096
F = T * TOPK
RB = 256
S = F + E * RB
NRB = S // RB
N1 = 1024
NN1 = I // N1
N2 = 2048
NN2 = H // N2

NC = 2
NS = 16
NW = NC * NS
LANES = 16
PW = S // NW
TW = T // NW

CH = 16
DBUF = 2
DCHUNKS = PW // CH
DROUNDS = DCHUNKS // DBUF

CCH = 8
CCHUNKS = TW // CCH
CROUNDS = CCHUNKS // 2

_SC_MESH = plsc.VectorSubcoreMesh(core_axis_name="c", subcore_axis_name="s")


def _dispatch_body(sp_hbm, x_hbm, ew_hbm, xs_hbm, ws_hbm,
                   sp_v, ws_v, *bufsem):
    wid = lax.axis_index("s") * NC + lax.axis_index("c")
    base = wid * PW
    pltpu.sync_copy(sp_hbm.at[pl.ds(base, PW)], sp_v)
    bufs = bufsem[:DBUF]
    sin = bufsem[DBUF:2 * DBUF]
    sout = bufsem[2 * DBUF:3 * DBUF]
    pltpu.async_copy(ew_hbm.at[sp_v], ws_v, sin[0]).wait()
    pltpu.sync_copy(ws_v, ws_hbm.at[pl.ds(base, PW)])

    def fire_in(b, c):
        spv = sp_v[pl.ds(c * CH, CH)]
        tok = lax.shift_right_logical(spv, 1)
        pltpu.async_copy(x_hbm.at[tok], bufs[b], sin[b])

    def wait_in(b):
        pltpu.make_async_copy(x_hbm.at[pl.ds(0, CH)], bufs[b], sin[b]).wait()

    def fire_out(b, c):
        pltpu.async_copy(bufs[b], xs_hbm.at[pl.ds(base + c * CH, CH)],
                         sout[b])

    def wait_out(b):
        pltpu.make_async_copy(bufs[b], xs_hbm.at[pl.ds(0, CH)],
                              sout[b]).wait()

    for b in range(DBUF):
        fire_in(b, b)

    def round_body(r, carry):
        for b in range(DBUF):
            wait_in(b)
            fire_out(b, r * DBUF + b)
        for b in range(DBUF):
            @pl.when(r < DROUNDS - 1)
            def _():
                wait_out(b)
                fire_in(b, (r + 1) * DBUF + b)
        return carry

    lax.fori_loop(0, DROUNDS, round_body, 0)
    for b in range(DBUF):
        wait_out(b)


_dispatch = pl.kernel(
    _dispatch_body,
    out_type=[
        jax.ShapeDtypeStruct((S, H), jnp.float32),
        jax.ShapeDtypeStruct((S,), jnp.float32),
    ],
    mesh=_SC_MESH,
    scratch_types=(
        [pltpu.VMEM((PW,), jnp.int32), pltpu.VMEM((PW,), jnp.float32)]
        + [pltpu.VMEM((CH, H), jnp.float32)] * DBUF
        + [pltpu.SemaphoreType.DMA] * (2 * DBUF)
    ),
)


def _combine_body(d0_hbm, d1_hbm, ys_hbm, out_hbm,
                  d0_v, d1_v, a0, b0, a1, b1, si0, si1, so0, so1):
    wid = lax.axis_index("s") * NC + lax.axis_index("c")
    base = wid * TW
    pltpu.sync_copy(d0_hbm.at[wid], d0_v)
    pltpu.sync_copy(d1_hbm.at[wid], d1_v)

    A = (a0, a1)
    B = (b0, b1)
    sin = (si0, si1)
    sout = (so0, so1)

    def fire_in(b, c):
        pltpu.async_copy(ys_hbm.at[d0_v.at[c]], A[b], sin[b])
        pltpu.async_copy(ys_hbm.at[d1_v.at[c]], B[b], sin[b])

    def wait_in(b):
        pltpu.make_async_copy(ys_hbm.at[pl.ds(0, CCH)], A[b], sin[b]).wait()
        pltpu.make_async_copy(ys_hbm.at[pl.ds(0, CCH)], B[b], sin[b]).wait()

    def compute(b):
        av, bv = A[b], B[b]

        def add16(k, carry2):
            r = k // (H // LANES)
            j = (k % (H // LANES)) * LANES
            av[r, pl.ds(j, LANES)] = (av[r, pl.ds(j, LANES)]
                                      + bv[r, pl.ds(j, LANES)])
            return carry2

        lax.fori_loop(0, CCH * (H // LANES), add16, 0)

    def fire_out(b, c):
        pltpu.async_copy(A[b], out_hbm.at[pl.ds(base + c * CCH, CCH)],
                         sout[b])

    def wait_out(b):
        pltpu.make_async_copy(A[b], out_hbm.at[pl.ds(0, CCH)],
                              sout[b]).wait()

    fire_in(0, 0)

    def round_body(r, carry):
        c = r * 2
        wait_in(0)

        @pl.when(r > 0)
        def _():
            wait_out(1)

        fire_in(1, c + 1)
        compute(0)
        fire_out(0, c)
        wait_in(1)

        @pl.when(r < CROUNDS - 1)
        def _():
            wait_out(0)
            fire_in(0, c + 2)

        compute(1)
        fire_out(1, c + 1)
        return carry

    lax.fori_loop(0, CROUNDS, round_body, 0)
    wait_out(0)
    wait_out(1)


_combine = pl.kernel(
    _combine_body,
    out_type=jax.ShapeDtypeStruct((T, H), jnp.float32),
    mesh=_SC_MESH,
    scratch_types=[
        pltpu.VMEM((CCHUNKS, CCH), jnp.int32),
        pltpu.VMEM((CCHUNKS, CCH), jnp.int32),
        pltpu.VMEM((CCH, H), jnp.float32),
        pltpu.VMEM((CCH, H), jnp.float32),
        pltpu.VMEM((CCH, H), jnp.float32),
        pltpu.VMEM((CCH, H), jnp.float32),
        pltpu.SemaphoreType.DMA,
        pltpu.SemaphoreType.DMA,
        pltpu.SemaphoreType.DMA,
        pltpu.SemaphoreType.DMA,
    ],
)


def _ffn1_body(be_ref, xs_ref, w1g_ref, w1u_ref, ws_ref, act_ref,
               w1g_s, w1u_s):
    i = pl.program_id(1)
    changed = (i == 0) | (be_ref[i] != be_ref[jnp.maximum(i - 1, 0)])

    @pl.when(changed)
    def _():
        w1g_s[...] = w1g_ref[0].astype(jnp.bfloat16)
        w1u_s[...] = w1u_ref[0].astype(jnp.bfloat16)

    xb = xs_ref[...].astype(jnp.bfloat16)
    g = lax.dot_general(xb, w1g_s[...], (((1,), (1,)), ((), ())),
                        preferred_element_type=jnp.float32)
    u = lax.dot_general(xb, w1u_s[...], (((1,), (1,)), ((), ())),
                        preferred_element_type=jnp.float32)
    act = (g * jax.nn.sigmoid(g)) * u * ws_ref[:, :1]
    act_ref[...] = act.astype(jnp.bfloat16)


def _ffn2_body(be_ref, act_ref, w2_ref, ys_ref, w2_s):
    i = pl.program_id(1)
    changed = (i == 0) | (be_ref[i] != be_ref[jnp.maximum(i - 1, 0)])

    @pl.when(changed)
    def _():
        w2_s[...] = w2_ref[0].astype(jnp.bfloat16)

    ys_ref[...] = lax.dot_general(act_ref[...], w2_s[...],
                                  (((1,), (1,)), ((), ())),
                                  preferred_element_type=jnp.float32)


_ffn1 = pl.pallas_call(
    _ffn1_body,
    grid_spec=pltpu.PrefetchScalarGridSpec(
        num_scalar_prefetch=1,
        grid=(NN1, NRB),
        in_specs=[
            pl.BlockSpec((RB, H), lambda n, i, be: (i, 0)),
            pl.BlockSpec((1, N1, H), lambda n, i, be: (be[i], n, 0)),
            pl.BlockSpec((1, N1, H), lambda n, i, be: (be[i], NN1 + n, 0)),
            pl.BlockSpec((RB, 128), lambda n, i, be: (i, 0)),
        ],
        out_specs=pl.BlockSpec((RB, N1), lambda n, i, be: (i, n)),
        scratch_shapes=[
            pltpu.VMEM((N1, H), jnp.bfloat16),
            pltpu.VMEM((N1, H), jnp.bfloat16),
        ],
    ),
    out_shape=jax.ShapeDtypeStruct((S, I), jnp.bfloat16),
)

_ffn2 = pl.pallas_call(
    _ffn2_body,
    grid_spec=pltpu.PrefetchScalarGridSpec(
        num_scalar_prefetch=1,
        grid=(NN2, NRB),
        in_specs=[
            pl.BlockSpec((RB, I), lambda n, i, be: (i, 0)),
            pl.BlockSpec((1, N2, I), lambda n, i, be: (be[i], n, 0)),
        ],
        out_specs=pl.BlockSpec((RB, N2), lambda n, i, be: (i, n)),
        scratch_shapes=[
            pltpu.VMEM((N2, I), jnp.bfloat16),
        ],
    ),
    out_shape=jax.ShapeDtypeStruct((S, H), jnp.float32),
)


def kernel(x, expert_weights, expert_indices, top_k, w1_weight, w2_weight):
    del top_k
    fe = expert_indices.reshape(F).astype(jnp.int32)
    oh = (fe[:, None] == jnp.arange(E, dtype=jnp.int32)[None, :]).astype(jnp.int32)
    csum = jnp.cumsum(oh, axis=0)
    counts = csum[-1]
    rank = jnp.sum((csum - oh) * oh, axis=1)
    pc = ((counts + RB - 1) // RB) * RB
    cum = jnp.cumsum(pc)
    poffs = cum - pc
    dest = poffs[fe] + rank
    sorted_pair = jnp.zeros((S,), jnp.int32).at[dest].set(
        jnp.arange(F, dtype=jnp.int32))
    block_expert = jnp.minimum(
        jnp.searchsorted(cum, jnp.arange(NRB, dtype=jnp.int32) * RB,
                         side="right"),
        E - 1).astype(jnp.int32)
    dtk = dest.reshape(T, TOPK)
    d0 = dtk[:, 0].astype(jnp.int32).reshape(NW, CCHUNKS, CCH)
    d1 = dtk[:, 1].astype(jnp.int32).reshape(NW, CCHUNKS, CCH)
    ewf = expert_weights.reshape(F).astype(jnp.float32)

    xs, ws = _dispatch(sorted_pair, x, ewf)
    wsb = jnp.broadcast_to(ws[:, None], (S, 128))
    act = _ffn1(block_expert, xs, w1_weight, w1_weight, wsb)
    ys = _ffn2(block_expert, act, w2_weight)
    return _combine(d0, d1, ys)


# speedup vs baseline: 1.9030x; 1.3321x over previous
import jax
import jax.numpy as jnp
from jax import lax
from jax.experimental import pallas as pl
from jax.experimental.pallas import tpu as pltpu
from jax.experimental.pallas import tpu_sc as plsc

E = 8
TOPK = 2
H = 2048
I = 1024
T = 4096
F = T * TOPK
RB = 256
S = F + E * RB
NRB = S // RB
N1 = 1024
NN1 = I // N1
N2 = 2048
NN2 = H // N2

NC = 2
NS = 16
NW = NC * NS
LANES = 16
PW = S // NW
TW = T // NW

CH = 16
DBUF = 2
DTW = T // NW
DCHUNKS = DTW // CH
DROUNDS = DCHUNKS // DBUF

CCH = 8
CCHUNKS = TW // CCH
CROUNDS = CCHUNKS // 2

_SC_MESH = plsc.VectorSubcoreMesh(core_axis_name="c", subcore_axis_name="s")


def _dispatch_body(d0_hbm, d1_hbm, dp_hbm, x_hbm, ew_hbm, xs_hbm, ws_hbm,
                   d0_v, d1_v, dp_v, ew_v, *bufsem):
    wid = lax.axis_index("s") * NC + lax.axis_index("c")
    tb = wid * DTW
    bufs = bufsem[:DBUF]
    sin = bufsem[DBUF:2 * DBUF]
    sout = bufsem[2 * DBUF:3 * DBUF]

    pltpu.sync_copy(d0_hbm.at[wid], d0_v)
    pltpu.sync_copy(d1_hbm.at[wid], d1_v)
    pltpu.sync_copy(dp_hbm.at[pl.ds(2 * tb, 2 * DTW)], dp_v)
    pltpu.sync_copy(ew_hbm.at[pl.ds(2 * tb, 2 * DTW)], ew_v)
    pltpu.async_copy(ew_v, ws_hbm.at[dp_v], sin[0]).wait()

    def fire_in(b, c):
        pltpu.async_copy(x_hbm.at[pl.ds(tb + c * CH, CH)], bufs[b], sin[b])

    def wait_in(b):
        pltpu.make_async_copy(x_hbm.at[pl.ds(0, CH)], bufs[b], sin[b]).wait()

    def fire_out(b, c):
        pltpu.async_copy(bufs[b], xs_hbm.at[d0_v.at[c]], sout[b])
        pltpu.async_copy(bufs[b], xs_hbm.at[d1_v.at[c]], sout[b])

    def wait_out(b):
        pltpu.make_async_copy(bufs[b], xs_hbm.at[pl.ds(0, CH)],
                              sout[b]).wait()
        pltpu.make_async_copy(bufs[b], xs_hbm.at[pl.ds(0, CH)],
                              sout[b]).wait()

    for b in range(DBUF):
        fire_in(b, b)

    def round_body(r, carry):
        for b in range(DBUF):
            wait_in(b)
            fire_out(b, r * DBUF + b)
        for b in range(DBUF):
            @pl.when(r < DROUNDS - 1)
            def _():
                wait_out(b)
                fire_in(b, (r + 1) * DBUF + b)
        return carry

    lax.fori_loop(0, DROUNDS, round_body, 0)
    for b in range(DBUF):
        wait_out(b)


_dispatch = pl.kernel(
    _dispatch_body,
    out_type=[
        jax.ShapeDtypeStruct((S, H), jnp.float32),
        jax.ShapeDtypeStruct((S,), jnp.float32),
    ],
    mesh=_SC_MESH,
    scratch_types=(
        [pltpu.VMEM((DCHUNKS, CH), jnp.int32),
         pltpu.VMEM((DCHUNKS, CH), jnp.int32),
         pltpu.VMEM((2 * DTW,), jnp.int32),
         pltpu.VMEM((2 * DTW,), jnp.float32)]
        + [pltpu.VMEM((CH, H), jnp.float32)] * DBUF
        + [pltpu.SemaphoreType.DMA] * (2 * DBUF)
    ),
)


def _combine_body(d0_hbm, d1_hbm, ys_hbm, out_hbm,
                  d0_v, d1_v, a0, b0, a1, b1, si0, si1, so0, so1):
    wid = lax.axis_index("s") * NC + lax.axis_index("c")
    base = wid * TW
    pltpu.sync_copy(d0_hbm.at[wid], d0_v)
    pltpu.sync_copy(d1_hbm.at[wid], d1_v)

    A = (a0, a1)
    B = (b0, b1)
    sin = (si0, si1)
    sout = (so0, so1)

    def fire_in(b, c):
        pltpu.async_copy(ys_hbm.at[d0_v.at[c]], A[b], sin[b])
        pltpu.async_copy(ys_hbm.at[d1_v.at[c]], B[b], sin[b])

    def wait_in(b):
        pltpu.make_async_copy(ys_hbm.at[pl.ds(0, CCH)], A[b], sin[b]).wait()
        pltpu.make_async_copy(ys_hbm.at[pl.ds(0, CCH)], B[b], sin[b]).wait()

    def compute(b):
        av, bv = A[b], B[b]

        def add16(k, carry2):
            r = k // (H // LANES)
            j = (k % (H // LANES)) * LANES
            av[r, pl.ds(j, LANES)] = (av[r, pl.ds(j, LANES)]
                                      + bv[r, pl.ds(j, LANES)])
            return carry2

        lax.fori_loop(0, CCH * (H // LANES), add16, 0)

    def fire_out(b, c):
        pltpu.async_copy(A[b], out_hbm.at[pl.ds(base + c * CCH, CCH)],
                         sout[b])

    def wait_out(b):
        pltpu.make_async_copy(A[b], out_hbm.at[pl.ds(0, CCH)],
                              sout[b]).wait()

    fire_in(0, 0)

    def round_body(r, carry):
        c = r * 2
        wait_in(0)

        @pl.when(r > 0)
        def _():
            wait_out(1)

        fire_in(1, c + 1)
        compute(0)
        fire_out(0, c)
        wait_in(1)

        @pl.when(r < CROUNDS - 1)
        def _():
            wait_out(0)
            fire_in(0, c + 2)

        compute(1)
        fire_out(1, c + 1)
        return carry

    lax.fori_loop(0, CROUNDS, round_body, 0)
    wait_out(0)
    wait_out(1)


_combine = pl.kernel(
    _combine_body,
    out_type=jax.ShapeDtypeStruct((T, H), jnp.float32),
    mesh=_SC_MESH,
    scratch_types=[
        pltpu.VMEM((CCHUNKS, CCH), jnp.int32),
        pltpu.VMEM((CCHUNKS, CCH), jnp.int32),
        pltpu.VMEM((CCH, H), jnp.float32),
        pltpu.VMEM((CCH, H), jnp.float32),
        pltpu.VMEM((CCH, H), jnp.float32),
        pltpu.VMEM((CCH, H), jnp.float32),
        pltpu.SemaphoreType.DMA,
        pltpu.SemaphoreType.DMA,
        pltpu.SemaphoreType.DMA,
        pltpu.SemaphoreType.DMA,
    ],
)


def _ffn1_body(be_ref, xs_ref, w1g_ref, w1u_ref, ws_ref, act_ref,
               w1g_s, w1u_s):
    i = pl.program_id(1)
    changed = (i == 0) | (be_ref[i] != be_ref[jnp.maximum(i - 1, 0)])

    @pl.when(changed)
    def _():
        w1g_s[...] = w1g_ref[0].astype(jnp.bfloat16)
        w1u_s[...] = w1u_ref[0].astype(jnp.bfloat16)

    xb = xs_ref[...].astype(jnp.bfloat16)
    g = lax.dot_general(xb, w1g_s[...], (((1,), (1,)), ((), ())),
                        preferred_element_type=jnp.float32)
    u = lax.dot_general(xb, w1u_s[...], (((1,), (1,)), ((), ())),
                        preferred_element_type=jnp.float32)
    act = (g * jax.nn.sigmoid(g)) * u * ws_ref[:, :1]
    act_ref[...] = act.astype(jnp.bfloat16)


def _ffn2_body(be_ref, act_ref, w2_ref, ys_ref, w2_s):
    i = pl.program_id(1)
    changed = (i == 0) | (be_ref[i] != be_ref[jnp.maximum(i - 1, 0)])

    @pl.when(changed)
    def _():
        w2_s[...] = w2_ref[0].astype(jnp.bfloat16)

    ys_ref[...] = lax.dot_general(act_ref[...], w2_s[...],
                                  (((1,), (1,)), ((), ())),
                                  preferred_element_type=jnp.float32)


_ffn1 = pl.pallas_call(
    _ffn1_body,
    grid_spec=pltpu.PrefetchScalarGridSpec(
        num_scalar_prefetch=1,
        grid=(NN1, NRB),
        in_specs=[
            pl.BlockSpec((RB, H), lambda n, i, be: (i, 0)),
            pl.BlockSpec((1, N1, H), lambda n, i, be: (be[i], n, 0)),
            pl.BlockSpec((1, N1, H), lambda n, i, be: (be[i], NN1 + n, 0)),
            pl.BlockSpec((RB, 128), lambda n, i, be: (i, 0)),
        ],
        out_specs=pl.BlockSpec((RB, N1), lambda n, i, be: (i, n)),
        scratch_shapes=[
            pltpu.VMEM((N1, H), jnp.bfloat16),
            pltpu.VMEM((N1, H), jnp.bfloat16),
        ],
    ),
    out_shape=jax.ShapeDtypeStruct((S, I), jnp.bfloat16),
)

_ffn2 = pl.pallas_call(
    _ffn2_body,
    grid_spec=pltpu.PrefetchScalarGridSpec(
        num_scalar_prefetch=1,
        grid=(NN2, NRB),
        in_specs=[
            pl.BlockSpec((RB, I), lambda n, i, be: (i, 0)),
            pl.BlockSpec((1, N2, I), lambda n, i, be: (be[i], n, 0)),
        ],
        out_specs=pl.BlockSpec((RB, N2), lambda n, i, be: (i, n)),
        scratch_shapes=[
            pltpu.VMEM((N2, I), jnp.bfloat16),
        ],
    ),
    out_shape=jax.ShapeDtypeStruct((S, H), jnp.float32),
)


def kernel(x, expert_weights, expert_indices, top_k, w1_weight, w2_weight):
    del top_k
    fe = expert_indices.reshape(F).astype(jnp.int32)
    oh = (fe[:, None] == jnp.arange(E, dtype=jnp.int32)[None, :]).astype(jnp.int32)
    csum = jnp.cumsum(oh, axis=0)
    counts = csum[-1]
    rank = jnp.sum((csum - oh) * oh, axis=1)
    pc = ((counts + RB - 1) // RB) * RB
    cum = jnp.cumsum(pc)
    poffs = cum - pc
    dest = poffs[fe] + rank
    block_expert = jnp.minimum(
        jnp.searchsorted(cum, jnp.arange(NRB, dtype=jnp.int32) * RB,
                         side="right"),
        E - 1).astype(jnp.int32)
    dtk = dest.reshape(T, TOPK)
    d0 = dtk[:, 0].astype(jnp.int32).reshape(NW, CCHUNKS, CCH)
    d1 = dtk[:, 1].astype(jnp.int32).reshape(NW, CCHUNKS, CCH)
    d0d = dtk[:, 0].astype(jnp.int32).reshape(NW, DCHUNKS, CH)
    d1d = dtk[:, 1].astype(jnp.int32).reshape(NW, DCHUNKS, CH)
    ewf = expert_weights.reshape(F).astype(jnp.float32)

    xs, ws = _dispatch(d0d, d1d, dest.astype(jnp.int32), x, ewf)
    wsb = jnp.broadcast_to(ws[:, None], (S, 128))
    act = _ffn1(block_expert, xs, w1_weight, w1_weight, wsb)
    ys = _ffn2(block_expert, act, w2_weight)
    return _combine(d0, d1, ys)
